# Initial kernel scaffold; baseline (speedup 1.0000x reference)
import jax
import jax.numpy as jnp
from jax.experimental import pallas as pl


def kernel(x, cp):
    raise NotImplementedError("write your pallas kernel here")


# speedup vs baseline: 2095.8095x; 2095.8095x over previous
import functools

import jax
import jax.numpy as jnp
from jax import lax
from jax.experimental import pallas as pl
from jax.experimental import pallas
from jax.experimental.pallas import tpu as pltpu
from jax.experimental.pallas import tpu_sc as plsc

B = 2048
N = 1024
C = 256
NUM_CORES = 2
NUM_SUBCORES = 16
NW = NUM_CORES * NUM_SUBCORES
ROWS_PER_W = B // NW
RB = 16
NCHUNK = ROWS_PER_W // RB
L = 16


def _sc_body(x_hbm, cp_hbm, o_hbm, xv, cpv, ov, sem_x, sem_cp, sem_o):
    wid = lax.axis_index("s") * NUM_CORES + lax.axis_index("c")
    row0 = wid * ROWS_PER_W

    @pl.loop(0, NCHUNK)
    def _chunk(ci):
        base = row0 + ci * RB
        cx = pltpu.async_copy(x_hbm.at[pl.ds(base, RB)], xv, sem_x)
        cc = pltpu.async_copy(cp_hbm.at[pl.ds(base, RB)], cpv, sem_cp)
        cx.wait()
        cc.wait()

        @pl.loop(0, RB)
        def _row(r):
            rvec = jnp.full((L,), r, dtype=jnp.int32)

            @pl.loop(0, N, step=L)
            def _grp(c0):
                xs = xv[r, pl.ds(c0, L)] * 254.0
                iv = jnp.clip(xs.astype(jnp.int32), 0, 254)
                u = xs - iv.astype(jnp.float32)
                u2 = u * u
                u3 = u2 * u
                om = 1.0 - u
                w0 = om * om * om * (1.0 / 6.0)
                w1 = (4.0 - 6.0 * u2 + 3.0 * u3) * (1.0 / 6.0)
                w2 = (1.0 + 3.0 * u + 3.0 * u2 - 3.0 * u3) * (1.0 / 6.0)
                w3 = u3 * (1.0 / 6.0)
                g0 = plsc.load_gather(cpv, [rvec, iv])
                g1 = plsc.load_gather(cpv, [rvec, iv + 1])
                g2 = plsc.load_gather(cpv, [rvec, jnp.minimum(iv + 2, 255)])
                g3 = plsc.load_gather(cpv, [rvec, jnp.minimum(iv + 3, 255)])
                ov[r, pl.ds(c0, L)] = w0 * g0 + w1 * g1 + w2 * g2 + w3 * g3

        pltpu.async_copy(ov, o_hbm.at[pl.ds(base, RB)], sem_o).wait()


@jax.jit
def _bspline_sc(x, cp):
    mesh = plsc.VectorSubcoreMesh(core_axis_name="c", subcore_axis_name="s")
    kern = functools.partial(
        pl.kernel,
        mesh=mesh,
        out_type=jax.ShapeDtypeStruct((B, N), jnp.float32),
        scratch_types=[
            pltpu.VMEM((RB, N), jnp.float32),
            pltpu.VMEM((RB, C), jnp.float32),
            pltpu.VMEM((RB, N), jnp.float32),
            pltpu.SemaphoreType.DMA,
            pltpu.SemaphoreType.DMA,
            pltpu.SemaphoreType.DMA,
        ],
        compiler_params=pltpu.CompilerParams(needs_layout_passes=False),
    )(_sc_body)
    return kern(x, cp)


def kernel(x, cp):
    return _bspline_sc(x, cp)
